# baseline (speedup 1.0000x reference)
import functools

import jax
import jax.numpy as jnp
from jax import lax
from jax.experimental import pallas as pl
from jax.experimental.pallas import tpu as pltpu
from jax.experimental.pallas import tpu_sc as plsc

D_MODEL = 64
D_PAD = 128
LANES = 128
NC = 2
NS = 16
NW = NC * NS


def _make_lookup(n_rows: int, n_b0: int):
    mesh = plsc.VectorSubcoreMesh(core_axis_name="c", subcore_axis_name="s")

    @functools.partial(
        pl.kernel,
        out_type=jax.ShapeDtypeStruct((n_rows, D_MODEL, n_b0), jnp.float32),
        mesh=mesh,
        scratch_types=(
            [pltpu.VMEM((n_rows, LANES), jnp.int32),
             pltpu.VMEM((2, LANES, D_PAD), jnp.float32),
             pltpu.VMEM((2, D_MODEL, LANES), jnp.float32)]
            + [pltpu.SemaphoreType.DMA] * 4
        ),
        compiler_params=pltpu.CompilerParams(use_tc_tiling_on_sc=True, needs_layout_passes=False),
    )
    def lookup(idx_hbm, tbl_hbm, out_hbm, idx_v, rows_v, outt_v, *sems):
        gsems, ssems = sems[:2], sems[2:]
        wid = lax.axis_index("s") * NC + lax.axis_index("c")
        lane0 = pl.multiple_of(wid * LANES, LANES)
        pltpu.sync_copy(idx_hbm.at[:, pl.ds(lane0, LANES)], idx_v)

        iota = lax.iota(jnp.int32, 16)

        def start_gather(b, j):
            pltpu.async_copy(tbl_hbm.at[idx_v.at[j]], rows_v.at[b], gsems[b])

        def wait_gather(b, j):
            pltpu.make_async_copy(
                tbl_hbm.at[idx_v.at[j]], rows_v.at[b], gsems[b]
            ).wait()

        def start_store(b, j):
            pltpu.async_copy(
                outt_v.at[b], out_hbm.at[j, :, pl.ds(lane0, LANES)], ssems[b]
            )

        def wait_store(b, j):
            pltpu.make_async_copy(
                outt_v.at[b], out_hbm.at[j, :, pl.ds(lane0, LANES)], ssems[b]
            ).wait()

        def transpose(b):
            rows = rows_v.at[b]
            outt = outt_v.at[b]

            def trow(r8, carry):
                for rr in range(8):
                    rfull = jnp.full((16,), r8 * 8 + rr, dtype=jnp.int32)
                    for k in range(D_MODEL // 16):
                        vals = plsc.load_gather(rows, [rfull, iota + (16 * k)])
                        plsc.store_scatter(outt, [iota + (16 * k), rfull], vals)
                return carry

            lax.fori_loop(0, LANES // 8, trow, 0)

        start_gather(0, 0)
        start_gather(1, 1)

        def body(g, carry):
            for b in range(2):
                j = 2 * g + b
                wait_gather(b, j)

                @pl.when(g >= 1)
                def _():
                    wait_store(b, j - 2)

                transpose(b)
                start_store(b, j)

                @pl.when(g + 1 < n_rows // 2)
                def _():
                    start_gather(b, j + 2)
            return carry

        lax.fori_loop(0, n_rows // 2, body, 0)
        wait_store(0, n_rows - 2)
        wait_store(1, n_rows - 1)

    return lookup


def kernel(x, table):
    b0, b1 = x.shape
    idx_t = x.T.astype(jnp.int32)
    tbl = jnp.pad(table, ((0, 0), (0, D_PAD - D_MODEL)))
    out_t = _make_lookup(b1, b0)(idx_t, tbl)
    return jnp.transpose(out_t, (2, 0, 1))


# speedup vs baseline: 1.5055x; 1.5055x over previous
import functools

import jax
import jax.numpy as jnp
from jax import lax
from jax.experimental import pallas as pl
from jax.experimental.pallas import tpu as pltpu
from jax.experimental.pallas import tpu_sc as plsc

D_MODEL = 64
D_PAD = 128
CHUNK = 128
NBUF = 4
NC = 2
NS = 16
NW = NC * NS


def _make_lookup(n_chunks: int):
    mesh = plsc.VectorSubcoreMesh(core_axis_name="c", subcore_axis_name="s")

    @functools.partial(
        pl.kernel,
        out_type=jax.ShapeDtypeStruct((NW, n_chunks, CHUNK, D_PAD), jnp.float32),
        mesh=mesh,
        scratch_types=(
            [pltpu.VMEM((n_chunks, CHUNK), jnp.int32),
             pltpu.VMEM((NBUF, CHUNK, D_PAD), jnp.float32)]
            + [pltpu.SemaphoreType.DMA] * (2 * NBUF)
        ),
        compiler_params=pltpu.CompilerParams(use_tc_tiling_on_sc=True),
    )
    def lookup(idx_hbm, tbl_hbm, out_hbm, idx_v, rows_v, *sems):
        gsems, ssems = sems[:NBUF], sems[NBUF:]
        wid = lax.axis_index("s") * NC + lax.axis_index("c")
        pltpu.sync_copy(idx_hbm.at[wid], idx_v)

        def start_gather(b, j):
            pltpu.async_copy(tbl_hbm.at[idx_v.at[j]], rows_v.at[b], gsems[b])

        def wait_gather(b, j):
            pltpu.make_async_copy(
                tbl_hbm.at[idx_v.at[j]], rows_v.at[b], gsems[b]
            ).wait()

        def start_store(b, j):
            pltpu.async_copy(rows_v.at[b], out_hbm.at[wid, j], ssems[b])

        def wait_store(b, j):
            pltpu.make_async_copy(
                rows_v.at[b], out_hbm.at[wid, j], ssems[b]
            ).wait()

        for b in range(NBUF):
            start_gather(b, b)

        def outer(g, carry):
            j0 = g * NBUF
            for b in range(NBUF):
                wait_gather(b, j0 + b)
                start_store(b, j0 + b)
            for b in range(NBUF):
                wait_store(b, j0 + b)
                start_gather(b, j0 + b + NBUF)
            return carry

        lax.fori_loop(0, n_chunks // NBUF - 1, outer, 0)

        j0 = n_chunks - NBUF
        for b in range(NBUF):
            wait_gather(b, j0 + b)
            start_store(b, j0 + b)
        for b in range(NBUF):
            wait_store(b, j0 + b)

    return lookup


def kernel(x, table):
    b0, b1 = x.shape
    total = b0 * b1
    n_chunks = total // (NW * CHUNK)
    idx = x.astype(jnp.int32).reshape(NW, n_chunks, CHUNK)
    tbl = jnp.pad(table, ((0, 0), (0, D_PAD - D_MODEL)))
    out = _make_lookup(n_chunks)(idx, tbl)
    return out.reshape(total, D_PAD)[:, :D_MODEL].reshape(b0, b1, D_MODEL)
